# baseline (speedup 1.0000x reference)
import functools

import jax
import jax.numpy as jnp
import numpy as np
from jax import lax
from jax.experimental import pallas as pl
from jax.experimental.pallas import tpu as pltpu
from jax.experimental.pallas import tpu_sc as plsc

N = 64
M = 16384
STATE = 8
CTRL = 4
OBS = 16


def _stage_a_body(obs_ref, ctrl_ref, at_ref, bt_ref, ct_ref,
                  spt_ref, noiset_ref, lwp_ref,
                  predt_ref, invw_ref, lwn_ref, est_ref):
    f = pl.program_id(0)
    spt = spt_ref[0]
    predt = jnp.dot(at_ref[:, :], spt, preferred_element_type=jnp.float32)
    ctrl = ctrl_ref[f, :]
    drive = jnp.dot(bt_ref[:, :], ctrl.reshape(CTRL, 1),
                    preferred_element_type=jnp.float32)
    predt = predt + drive + noiset_ref[0]
    pobst = jnp.dot(ct_ref[:, :], predt, preferred_element_type=jnp.float32)
    err = obs_ref[f, :].reshape(OBS, 1) - pobst
    ll = -0.5 * jnp.sum(err * err, axis=0, keepdims=True)
    lw = lwp_ref[0] + ll
    mx = jnp.max(lw)
    lse = jnp.log(jnp.sum(jnp.exp(lw - mx))) + mx
    lwn = lw - lse
    w = jnp.exp(lwn)
    est = jnp.sum(w * predt, axis=1, keepdims=True)
    predt_ref[0] = predt
    invw_ref[0] = -jnp.exp(-lwn)
    lwn_ref[0] = lwn
    est_ref[0] = est.reshape(1, STATE)


def _stage_a(states_prev, log_weights_prev, observations, controls, A, B, C, noise):
    grid = (N,)
    out_shapes = (
        jax.ShapeDtypeStruct((N, STATE, M), jnp.float32),
        jax.ShapeDtypeStruct((N, 1, M), jnp.float32),
        jax.ShapeDtypeStruct((N, 1, M), jnp.float32),
        jax.ShapeDtypeStruct((N, 1, STATE), jnp.float32),
    )
    full = lambda shape: pl.BlockSpec(shape, lambda f: tuple(0 for _ in shape))
    predt, invw, lwn, est = pl.pallas_call(
        _stage_a_body,
        grid=grid,
        in_specs=[
            full((N, OBS)),
            full((N, CTRL)),
            full((STATE, STATE)),
            full((STATE, CTRL)),
            full((OBS, STATE)),
            pl.BlockSpec((1, STATE, M), lambda f: (f, 0, 0)),
            pl.BlockSpec((1, STATE, M), lambda f: (f, 0, 0)),
            pl.BlockSpec((1, 1, M), lambda f: (f, 0, 0)),
        ],
        out_specs=(
            pl.BlockSpec((1, STATE, M), lambda f: (f, 0, 0)),
            pl.BlockSpec((1, 1, M), lambda f: (f, 0, 0)),
            pl.BlockSpec((1, 1, M), lambda f: (f, 0, 0)),
            pl.BlockSpec((1, 1, STATE), lambda f: (f, 0, 0)),
        ),
        out_shape=out_shapes,
    )(observations, controls, A.T, B.T, C.T,
      states_prev.transpose(0, 2, 1), noise.transpose(0, 2, 1),
      log_weights_prev.reshape(N, 1, M))
    return (predt.transpose(0, 2, 1), invw,
            lwn.reshape(N, M), est.reshape(N, STATE))


_SB = 64
_CC = 2048
_NSB = M // _SB
_NC = M // _CC
_TINY = float(np.finfo(np.float32).tiny)

_ROTS = ((13, 15, 26, 6), (17, 29, 16, 24))


def _rotl(x, r):
    return lax.shift_left(x, r) | lax.shift_right_logical(x, 32 - r)


def _threefry_bits(hi, lo7):
    k0 = jnp.int32(0)
    k1 = jnp.int32(7)
    k2 = jnp.int32(0x1BD11BDD)
    ks = (k1, k2, k0)
    x0 = hi
    x1 = lo7
    for i in range(5):
        for r in _ROTS[i % 2]:
            x0 = x0 + x1
            x1 = _rotl(x1, r) ^ x0
        x0 = x0 + ks[i % 3]
        x1 = x1 + ks[(i + 1) % 3] + jnp.int32(i + 1)
    return x0 ^ x1


def _sampler_body(ninvw_ref, idx_ref):
    f = pl.program_id(0)
    sb = pl.program_id(1)
    s = sb * _SB + lax.broadcasted_iota(jnp.int32, (_SB, 1), 0)
    hi = lax.shift_right_logical(s, 12)
    lo_base = lax.shift_left(s & 0xFFF, 20) | lax.shift_left(f, 14)
    lane = lax.broadcasted_iota(jnp.int32, (_SB, _CC), 1)
    lobl = (lo_base | lane) + 7

    def chunk(c0, carry):
        best_val, best_idx = carry
        bits = _threefry_bits(hi, lobl + c0 * _CC)
        fbits = lax.shift_right_logical(bits, 9) | jnp.int32(0x3F800000)
        u = lax.bitcast_convert_type(fbits, jnp.float32) - jnp.float32(1.0)
        ninvw = ninvw_ref[0, 0, pl.ds(c0 * _CC, _CC)].reshape(1, _CC)
        r = jnp.log(u) * ninvw
        cmin = jnp.min(r, axis=1, keepdims=True)
        cidx = jnp.min(jnp.where(r <= cmin, lane, jnp.int32(M)),
                       axis=1, keepdims=True)
        upd = cmin < best_val
        return (jnp.where(upd, cmin, best_val),
                jnp.where(upd, cidx + c0 * _CC, best_idx))

    init = (jnp.full((_SB, 1), jnp.inf, jnp.float32),
            jnp.zeros((_SB, 1), jnp.int32))
    _, best_idx = lax.fori_loop(0, _NC, chunk, init)
    idx_ref[0, 0] = best_idx + f * M


_SSC = 3648
_S0 = M - _SSC
_NSB_TC = _S0 // _SB


def _sampler(invw3):
    return pl.pallas_call(
        _sampler_body,
        grid=(N, _NSB_TC),
        in_specs=[pl.BlockSpec((1, 1, M), lambda f, sb: (f, 0, 0))],
        out_specs=pl.BlockSpec((1, 1, _SB, 1), lambda f, sb: (f, sb, 0, 0)),
        out_shape=jax.ShapeDtypeStruct((N, _NSB_TC, _SB, 1), jnp.int32),
    )(invw3).reshape(N, _S0)


def _sc_ln(u):
    ib = lax.bitcast_convert_type(u, jnp.int32)
    e = lax.shift_right_logical(ib, 23) - 127
    mm = lax.bitcast_convert_type((ib & 0x7FFFFF) | jnp.int32(0x3F800000),
                                  jnp.float32)
    x = mm - 1.0
    y = x * (jnp.float32(0.9973) + x * jnp.float32(-0.3045))
    for _ in range(2):
        y = y + mm * jnp.exp(-y) - 1.0
    ln_gen = e.astype(jnp.float32) * jnp.float32(0.6931471805599453) + y
    d = 1.0 - u
    p = jnp.full((16,), 1.0 / 7.0, jnp.float32)
    for k in (6, 5, 4, 3, 2):
        p = p * d + jnp.float32(1.0 / k)
    ser = -(d * (1.0 + d * p))
    return jnp.where(d < 0.125, ser, ln_gen)


def _sc_sampler_body(ninvw_hbm, out_hbm, niv_v, out_v):
    wid = lax.axis_index("s") * 2 + lax.axis_index("c")
    iota = lax.iota(jnp.int32, 16)
    for fo in range(2):
        f = wid * 2 + fo
        pltpu.sync_copy(ninvw_hbm.at[pl.ds(f * M, M)], niv_v)

        def sgroup(sg, _):
            lov = (lax.shift_left((_S0 & 0xFFF) + sg * 16 + iota, 20)
                   | lax.shift_left(f, 14)) + 7

            def cgroup(g, carry):
                rbest, cbest = carry
                nivvec = niv_v[pl.ds(g * 16, 16)]
                base = g * 16
                for j in range(16):
                    c = base + j
                    bits = _threefry_bits(jnp.int32(3), lov + c)
                    fb = (lax.shift_right_logical(bits, 9)
                          | jnp.int32(0x3F800000))
                    u = lax.bitcast_convert_type(fb, jnp.float32) - 1.0
                    rr = _sc_ln(u) * nivvec[j]
                    m = rr < rbest
                    rbest = jnp.where(m, rr, rbest)
                    cbest = jnp.where(m, c, cbest)
                return rbest, cbest

            rbest, cbest = lax.fori_loop(
                0, M // 16, cgroup,
                (jnp.full((16,), jnp.inf, jnp.float32),
                 jnp.zeros((16,), jnp.int32)))
            out_v[pl.ds(sg * 16, 16)] = cbest + f * M
            return 0

        lax.fori_loop(0, _SSC // 16, sgroup, 0)
        pltpu.sync_copy(out_v, out_hbm.at[pl.ds(f * _SSC, _SSC)])


def _sc_sampler(ninvw_flat):
    mesh = plsc.VectorSubcoreMesh(core_axis_name="c", subcore_axis_name="s")
    fn = pl.kernel(
        _sc_sampler_body,
        mesh=mesh,
        compiler_params=pltpu.CompilerParams(use_tc_tiling_on_sc=False),
        out_type=jax.ShapeDtypeStruct((N * _SSC,), jnp.int32),
        scratch_types=[
            pltpu.VMEM((M,), jnp.float32),
            pltpu.VMEM((_SSC,), jnp.int32),
        ],
    )
    return fn(ninvw_flat)


_NW = 32
_BW = (N * M) // _NW
_GCK = 2048
_GNC = _BW // _GCK


def _gather_body(table_hbm, gidx_hbm, out_hbm, idx_v, rows_v, sem):
    wid = lax.axis_index("s") * 2 + lax.axis_index("c")

    def chunk(ci, _):
        base = wid * _BW + ci * _GCK
        pltpu.sync_copy(gidx_hbm.at[pl.ds(base, _GCK)], idx_v)
        pltpu.async_copy(table_hbm.at[idx_v], rows_v, sem).wait()
        pltpu.sync_copy(rows_v, out_hbm.at[pl.ds(base, _GCK)])
        return 0

    lax.fori_loop(0, _GNC, chunk, 0, unroll=False)


def _sc_gather(table, gidx_flat):
    mesh = plsc.VectorSubcoreMesh(core_axis_name="c", subcore_axis_name="s")
    fn = pl.kernel(
        _gather_body,
        mesh=mesh,
        compiler_params=pltpu.CompilerParams(use_tc_tiling_on_sc=False),
        out_type=jax.ShapeDtypeStruct((N * M, STATE), jnp.float32),
        scratch_types=[
            pltpu.VMEM((_GCK,), jnp.int32),
            pltpu.VMEM((_GCK, STATE), jnp.float32),
            pltpu.SemaphoreType.DMA,
        ],
    )
    return fn(table, gidx_flat)


def kernel(states_prev, log_weights_prev, observations, controls, A, B, C):
    n, m, state_dim = states_prev.shape
    noise = jax.random.normal(jax.random.key(42), (n, m, state_dim),
                              dtype=jnp.float32) * 0.01
    states_pred, invw3, lwn, state_estimates = _stage_a(
        states_prev, log_weights_prev, observations, controls, A, B, C, noise)
    gidx_tc = _sampler(invw3)
    gidx_sc = _sc_sampler(invw3.reshape(-1)).reshape(n, _SSC)
    gidx = jnp.concatenate([gidx_tc, gidx_sc], axis=1)
    states = _sc_gather(states_pred.reshape(n * m, state_dim),
                        gidx.reshape(-1)).reshape(n, m, state_dim)
    log_weights = jnp.full((n, m), -float(np.log(m)), dtype=jnp.float32)
    return (state_estimates, states, log_weights)


# speedup vs baseline: 1.0051x; 1.0051x over previous
import jax
import jax.numpy as jnp
import numpy as np
from jax import lax
from jax.experimental import pallas as pl
from jax.experimental.pallas import tpu as pltpu
from jax.experimental.pallas import tpu_sc as plsc

N = 64
M = 16384
STATE = 8
CTRL = 4
OBS = 16


def _stage_a_body(obs_ref, ctrl_ref, at_ref, bt_ref, ct_ref,
                  spt_ref, noiset_ref, lwp_ref,
                  predt_ref, invw_ref, lwn_ref, est_ref):
    f = pl.program_id(0)
    spt = spt_ref[0]
    predt = jnp.dot(at_ref[:, :], spt, preferred_element_type=jnp.float32)
    ctrl = ctrl_ref[f, :]
    drive = jnp.dot(bt_ref[:, :], ctrl.reshape(CTRL, 1),
                    preferred_element_type=jnp.float32)
    predt = predt + drive + noiset_ref[0]
    pobst = jnp.dot(ct_ref[:, :], predt, preferred_element_type=jnp.float32)
    err = obs_ref[f, :].reshape(OBS, 1) - pobst
    ll = -0.5 * jnp.sum(err * err, axis=0, keepdims=True)
    lw = lwp_ref[0] + ll
    mx = jnp.max(lw)
    lse = jnp.log(jnp.sum(jnp.exp(lw - mx))) + mx
    lwn = lw - lse
    w = jnp.exp(lwn)
    est = jnp.sum(w * predt, axis=1, keepdims=True)
    predt_ref[0] = predt
    invw_ref[0] = -jnp.exp(-lwn)
    lwn_ref[0] = lwn
    est_ref[0] = est.reshape(1, STATE)


def _stage_a(states_prev, log_weights_prev, observations, controls, A, B, C, noise):
    grid = (N,)
    out_shapes = (
        jax.ShapeDtypeStruct((N, STATE, M), jnp.float32),
        jax.ShapeDtypeStruct((N, 1, M), jnp.float32),
        jax.ShapeDtypeStruct((N, 1, M), jnp.float32),
        jax.ShapeDtypeStruct((N, 1, STATE), jnp.float32),
    )
    full = lambda shape: pl.BlockSpec(shape, lambda f: tuple(0 for _ in shape))
    predt, invw, lwn, est = pl.pallas_call(
        _stage_a_body,
        grid=grid,
        in_specs=[
            full((N, OBS)),
            full((N, CTRL)),
            full((STATE, STATE)),
            full((STATE, CTRL)),
            full((OBS, STATE)),
            pl.BlockSpec((1, STATE, M), lambda f: (f, 0, 0)),
            pl.BlockSpec((1, STATE, M), lambda f: (f, 0, 0)),
            pl.BlockSpec((1, 1, M), lambda f: (f, 0, 0)),
        ],
        out_specs=(
            pl.BlockSpec((1, STATE, M), lambda f: (f, 0, 0)),
            pl.BlockSpec((1, 1, M), lambda f: (f, 0, 0)),
            pl.BlockSpec((1, 1, M), lambda f: (f, 0, 0)),
            pl.BlockSpec((1, 1, STATE), lambda f: (f, 0, 0)),
        ),
        out_shape=out_shapes,
    )(observations, controls, A.T, B.T, C.T,
      states_prev.transpose(0, 2, 1), noise.transpose(0, 2, 1),
      log_weights_prev.reshape(N, 1, M))
    return (predt.transpose(0, 2, 1), invw,
            lwn.reshape(N, M), est.reshape(N, STATE))


_SB = 64
_CC = 2048
_NC = M // _CC

_ROTS = ((13, 15, 26, 6), (17, 29, 16, 24))


def _rotl(x, r):
    return lax.shift_left(x, r) | lax.shift_right_logical(x, 32 - r)


def _threefry_bits(hi, lo7):
    k0 = jnp.int32(0)
    k1 = jnp.int32(7)
    k2 = jnp.int32(0x1BD11BDD)
    ks = (k1, k2, k0)
    x0 = hi
    x1 = lo7
    for i in range(5):
        for r in _ROTS[i % 2]:
            x0 = x0 + x1
            x1 = _rotl(x1, r) ^ x0
        x0 = x0 + ks[i % 3]
        x1 = x1 + ks[(i + 1) % 3] + jnp.int32(i + 1)
    return x0 ^ x1


def _sampler_body(ninvw_ref, idx_ref):
    f = pl.program_id(0)
    sb = pl.program_id(1)
    s = sb * _SB + lax.broadcasted_iota(jnp.int32, (_SB, 1), 0)
    hi = lax.shift_right_logical(s, 12)
    lo_base = lax.shift_left(s & 0xFFF, 20) | lax.shift_left(f, 14)
    lane = lax.broadcasted_iota(jnp.int32, (_SB, _CC), 1)
    lobl = (lo_base | lane) + 7

    def chunk(c0, carry):
        best_val, best_idx = carry
        bits = _threefry_bits(hi, lobl + c0 * _CC)
        fbits = lax.shift_right_logical(bits, 9) | jnp.int32(0x3F800000)
        u = lax.bitcast_convert_type(fbits, jnp.float32) - jnp.float32(1.0)
        ninvw = ninvw_ref[0, 0, pl.ds(c0 * _CC, _CC)].reshape(1, _CC)
        r = jnp.log(u) * ninvw
        cmin = jnp.min(r, axis=1, keepdims=True)
        cidx = jnp.min(jnp.where(r <= cmin, lane, jnp.int32(M)),
                       axis=1, keepdims=True)
        upd = cmin < best_val
        return (jnp.where(upd, cmin, best_val),
                jnp.where(upd, cidx + c0 * _CC, best_idx))

    init = (jnp.full((_SB, 1), jnp.inf, jnp.float32),
            jnp.zeros((_SB, 1), jnp.int32))
    _, best_idx = lax.fori_loop(0, _NC, chunk, init)
    idx_ref[0, 0] = best_idx + f * M


_SSC = 3712
_S0 = M - _SSC
_NSB_TC = _S0 // _SB


def _sampler(invw3):
    return pl.pallas_call(
        _sampler_body,
        grid=(N, _NSB_TC),
        in_specs=[pl.BlockSpec((1, 1, M), lambda f, sb: (f, 0, 0))],
        out_specs=pl.BlockSpec((1, 1, _SB, 1), lambda f, sb: (f, sb, 0, 0)),
        out_shape=jax.ShapeDtypeStruct((N, _NSB_TC, _SB, 1), jnp.int32),
    )(invw3).reshape(N, _S0)


def _sc_ln(u):
    ib = lax.bitcast_convert_type(u, jnp.int32)
    e = lax.shift_right_logical(ib, 23) - 127
    mm = lax.bitcast_convert_type((ib & 0x7FFFFF) | jnp.int32(0x3F800000),
                                  jnp.float32)
    x = mm - 1.0
    y = x * (jnp.float32(0.9973) + x * jnp.float32(-0.3045))
    for _ in range(2):
        y = y + mm * jnp.exp(-y) - 1.0
    ln_gen = e.astype(jnp.float32) * jnp.float32(0.6931471805599453) + y
    d = 1.0 - u
    p = jnp.full((16,), 1.0 / 7.0, jnp.float32)
    for k in (6, 5, 4, 3, 2):
        p = p * d + jnp.float32(1.0 / k)
    ser = -(d * (1.0 + d * p))
    return jnp.where(d < 0.125, ser, ln_gen)


def _sc_sampler_body(ninvw_hbm, out_hbm, niv_v, out_v):
    wid = lax.axis_index("s") * 2 + lax.axis_index("c")
    iota = lax.iota(jnp.int32, 16)
    for fo in range(2):
        f = wid * 2 + fo
        pltpu.sync_copy(ninvw_hbm.at[pl.ds(f * M, M)], niv_v)

        def sgroup(sg, _):
            lov = (lax.shift_left((_S0 & 0xFFF) + sg * 16 + iota, 20)
                   | lax.shift_left(f, 14)) + 7

            def cgroup(g, carry):
                rbest, cbest = carry
                nivvec = niv_v[pl.ds(g * 16, 16)]
                base = g * 16
                for j in range(16):
                    c = base + j
                    bits = _threefry_bits(jnp.int32(3), lov + c)
                    fb = (lax.shift_right_logical(bits, 9)
                          | jnp.int32(0x3F800000))
                    u = lax.bitcast_convert_type(fb, jnp.float32) - 1.0
                    rr = _sc_ln(u) * nivvec[j]
                    m = rr < rbest
                    rbest = jnp.where(m, rr, rbest)
                    cbest = jnp.where(m, c, cbest)
                return rbest, cbest

            rbest, cbest = lax.fori_loop(
                0, M // 16, cgroup,
                (jnp.full((16,), jnp.inf, jnp.float32),
                 jnp.zeros((16,), jnp.int32)))
            out_v[pl.ds(sg * 16, 16)] = cbest + f * M
            return 0

        lax.fori_loop(0, _SSC // 16, sgroup, 0)
        pltpu.sync_copy(out_v, out_hbm.at[pl.ds(f * _SSC, _SSC)])


def _sc_sampler(ninvw_flat):
    mesh = plsc.VectorSubcoreMesh(core_axis_name="c", subcore_axis_name="s")
    fn = pl.kernel(
        _sc_sampler_body,
        mesh=mesh,
        compiler_params=pltpu.CompilerParams(use_tc_tiling_on_sc=False),
        out_type=jax.ShapeDtypeStruct((N * _SSC,), jnp.int32),
        scratch_types=[
            pltpu.VMEM((M,), jnp.float32),
            pltpu.VMEM((_SSC,), jnp.int32),
        ],
    )
    return fn(ninvw_flat)


_NW = 32
_BW = (N * M) // _NW
_GCK = 2048
_GNC = _BW // _GCK


def _gather_body(table_hbm, gidx_hbm, out_hbm, idx_v, rows_v, sem):
    wid = lax.axis_index("s") * 2 + lax.axis_index("c")

    def chunk(ci, _):
        base = wid * _BW + ci * _GCK
        pltpu.sync_copy(gidx_hbm.at[pl.ds(base, _GCK)], idx_v)
        pltpu.async_copy(table_hbm.at[idx_v], rows_v, sem).wait()
        pltpu.sync_copy(rows_v, out_hbm.at[pl.ds(base, _GCK)])
        return 0

    lax.fori_loop(0, _GNC, chunk, 0, unroll=False)


def _sc_gather(table, gidx_flat):
    mesh = plsc.VectorSubcoreMesh(core_axis_name="c", subcore_axis_name="s")
    fn = pl.kernel(
        _gather_body,
        mesh=mesh,
        compiler_params=pltpu.CompilerParams(use_tc_tiling_on_sc=False),
        out_type=jax.ShapeDtypeStruct((N * M, STATE), jnp.float32),
        scratch_types=[
            pltpu.VMEM((_GCK,), jnp.int32),
            pltpu.VMEM((_GCK, STATE), jnp.float32),
            pltpu.SemaphoreType.DMA,
        ],
    )
    return fn(table, gidx_flat)


def kernel(states_prev, log_weights_prev, observations, controls, A, B, C):
    n, m, state_dim = states_prev.shape
    noise = jax.random.normal(jax.random.key(42), (n, m, state_dim),
                              dtype=jnp.float32) * 0.01
    states_pred, invw3, lwn, state_estimates = _stage_a(
        states_prev, log_weights_prev, observations, controls, A, B, C, noise)
    gidx_tc = _sampler(invw3)
    gidx_sc = _sc_sampler(invw3.reshape(-1)).reshape(n, _SSC)
    gidx = jnp.concatenate([gidx_tc, gidx_sc], axis=1)
    states = _sc_gather(states_pred.reshape(n * m, state_dim),
                        gidx.reshape(-1)).reshape(n, m, state_dim)
    log_weights = jnp.full((n, m), -float(np.log(m)), dtype=jnp.float32)
    return (state_estimates, states, log_weights)
